# baseline (speedup 1.0000x reference)
import functools

import jax
import jax.numpy as jnp
from jax import lax
from jax.experimental import pallas as pl
from jax.experimental.pallas import tpu as pltpu
from jax.experimental.pallas import tpu_sc as plsc

M = 16384
B = 16384
D = 2324
DC = 64
N_FULL = 36
D_MAIN = N_FULL * DC
TAIL = D - D_MAIN
TPAD = 24
RPS = M // 16
SPS = B // 16
SB = 256
NR = SPS // SB
NGR = SB // 128


def _sc_scatter(idx2d, span_emb, span_tail):
    mesh = plsc.VectorSubcoreMesh(core_axis_name="c", subcore_axis_name="s")

    @functools.partial(
        pl.kernel,
        mesh=mesh,
        compiler_params=pltpu.CompilerParams(use_tc_tiling_on_sc=False),
        out_type=[
            jax.ShapeDtypeStruct((M, TPAD), jnp.float32),
            jax.ShapeDtypeStruct((M, D_MAIN), jnp.float32),
            jax.ShapeDtypeStruct((M, TPAD), jnp.float32),
        ],
        scratch_types=[
            pltpu.VMEM_SHARED((M, DC), jnp.float32),
            pltpu.VMEM_SHARED((M, TPAD), jnp.float32),
            pltpu.VMEM((SB, DC), jnp.float32),
            pltpu.VMEM((SB, TPAD), jnp.float32),
            pltpu.VMEM((128, DC), jnp.float32),
            pltpu.VMEM((128, TPAD), jnp.float32),
            pltpu.VMEM((NR * NGR, 128), jnp.int32),
        ],
    )
    def k(idx_hbm, span_hbm, tail_hbm, cnt_hbm, sums_hbm, tsums_hbm,
          acc, acc_misc, sbuf, sbuf_tail, zbuf, ones_b, idxb):
        cid = lax.axis_index("c")
        sid = lax.axis_index("s")
        r0 = sid * RPS
        b0 = sid * SPS

        zv = jnp.zeros((16,), jnp.float32)
        ov = jnp.ones((16,), jnp.float32)

        def zrow(i, carry):
            for j in range(DC // 16):
                zbuf[i, pl.ds(j * 16, 16)] = zv
            return carry
        lax.fori_loop(0, 128, zrow, 0)

        def orow(i, carry):
            ones_b[i, pl.ds(0, 16)] = ov
            ones_b[i, pl.ds(8, 16)] = ov
            return carry
        lax.fori_loop(0, 128, orow, 0)

        pltpu.sync_copy(idx_hbm.at[pl.ds(sid * NR * NGR, NR * NGR)], idxb)

        @pl.when(cid == 0)
        def _():
            for t in range(RPS // 128):
                pltpu.sync_copy(zbuf.at[:, pl.ds(0, TPAD)],
                                acc_misc.at[pl.ds(r0 + t * 128, 128)])
            plsc.subcore_barrier()
            for q in range(NR):
                pltpu.sync_copy(tail_hbm.at[pl.ds(b0 + q * SB, SB)], sbuf_tail)
                for g in range(NGR):
                    pltpu.sync_copy(sbuf_tail.at[pl.ds(g * 128, 128)],
                                    acc_misc.at[idxb.at[q * NGR + g]], add=True)
            plsc.subcore_barrier()
            pltpu.sync_copy(acc_misc.at[pl.ds(r0, RPS)], tsums_hbm.at[pl.ds(r0, RPS)])

        @pl.when(cid == 1)
        def _():
            for t in range(RPS // 128):
                pltpu.sync_copy(zbuf.at[:, pl.ds(0, TPAD)],
                                acc_misc.at[pl.ds(r0 + t * 128, 128)])
            plsc.subcore_barrier()
            for q in range(NR):
                for g in range(NGR):
                    pltpu.sync_copy(ones_b, acc_misc.at[idxb.at[q * NGR + g]],
                                    add=True)
            plsc.subcore_barrier()
            pltpu.sync_copy(acc_misc.at[pl.ds(r0, RPS)], cnt_hbm.at[pl.ds(r0, RPS)])

        def chunk_body(i, carry):
            c0 = (i * 2 + cid) * DC
            for t in range(RPS // 128):
                pltpu.sync_copy(zbuf, acc.at[pl.ds(r0 + t * 128, 128)])
            plsc.subcore_barrier()
            for q in range(NR):
                pltpu.sync_copy(
                    span_hbm.at[pl.ds(b0 + q * SB, SB), pl.ds(c0, DC)], sbuf)
                for g in range(NGR):
                    pltpu.sync_copy(sbuf.at[pl.ds(g * 128, 128)],
                                    acc.at[idxb.at[q * NGR + g]], add=True)
            plsc.subcore_barrier()
            pltpu.sync_copy(acc.at[pl.ds(r0, RPS)],
                            sums_hbm.at[pl.ds(r0, RPS), pl.ds(c0, DC)])
            return carry
        lax.fori_loop(0, N_FULL // 2, chunk_body, 0)

    return k(idx2d, span_emb, span_tail)


def _tc_combine(cluster_emb, sizes2d, counts24, sums_main, sums_tail):
    R = 512

    def body(c_ref, s_ref, n_ref, m_ref, t_ref, o_ref):
        sf = s_ref[...]
        cnt = n_ref[:, 0:1]
        inv = 1.0 / (sf + cnt)
        o_ref[:, :D_MAIN] = (c_ref[:, :D_MAIN] * sf + m_ref[...]) * inv
        o_ref[:, D_MAIN:D] = (c_ref[:, D_MAIN:D] * sf + t_ref[:, :TAIL]) * inv

    return pl.pallas_call(
        body,
        grid=(M // R,),
        in_specs=[
            pl.BlockSpec((R, D), lambda i: (i, 0)),
            pl.BlockSpec((R, 1), lambda i: (i, 0)),
            pl.BlockSpec((R, TPAD), lambda i: (i, 0)),
            pl.BlockSpec((R, D_MAIN), lambda i: (i, 0)),
            pl.BlockSpec((R, TPAD), lambda i: (i, 0)),
        ],
        out_specs=pl.BlockSpec((R, D), lambda i: (i, 0)),
        out_shape=jax.ShapeDtypeStruct((M, D), jnp.float32),
    )(cluster_emb, sizes2d, counts24, sums_main, sums_tail)


def kernel(cluster_emb, cluster_sizes, idx, span_emb):
    idx2d = idx.astype(jnp.int32).reshape(128, 128)
    span_tail = jnp.pad(span_emb[:, D_MAIN:], ((0, 0), (0, TPAD - TAIL)))
    counts24, sums_main, sums_tail = _sc_scatter(idx2d, span_emb, span_tail)
    return _tc_combine(cluster_emb, cluster_sizes.reshape(M, 1).astype(jnp.float32),
                       counts24, sums_main, sums_tail)
